# Initial kernel scaffold; baseline (speedup 1.0000x reference)
import functools

import jax
import jax.numpy as jnp
from jax import lax
from jax.experimental import pallas as pl
from jax.experimental.pallas import tpu as pltpu
from jax.experimental.pallas import tpu_sc as plsc

N_AGT = 10000
N_CTX = 10000
E = 160000
D = 128

NC = 2
NS = 16
NW = NC * NS
CHUNK = 128
NCHUNK = E // CHUNK
RB = 500
BE = 3200
_EPS = 1e-5


def _gn(x, g, b):
    m = jnp.mean(x, axis=1, keepdims=True)
    xc = x - m
    v = jnp.mean(xc * xc, axis=1, keepdims=True)
    return xc * lax.rsqrt(v + _EPS) * g + b


def _node_pre_body(agts_ref, ctx_ref, WqT, gq, bq, WBT, WCT, WaT,
                   qb_ref, cc_ref, awa_ref):
    x = agts_ref[...]
    q = jnp.dot(x, WqT[...], preferred_element_type=jnp.float32)
    q = jnp.maximum(_gn(q, gq[...], bq[...]), 0.0)
    qb_ref[...] = jnp.dot(q, WBT[...], preferred_element_type=jnp.float32)
    cc_ref[...] = jnp.dot(ctx_ref[...], WCT[...],
                          preferred_element_type=jnp.float32)
    awa_ref[...] = jnp.dot(x, WaT[...], preferred_element_type=jnp.float32)


def _node_pre(agts, ctx, WqT, gq, bq, WBT, WCT, WaT):
    grid = (N_AGT // RB,)
    row = pl.BlockSpec((RB, D), lambda i: (i, 0))
    full = pl.BlockSpec((D, D), lambda i: (0, 0))
    vec = pl.BlockSpec((1, D), lambda i: (0, 0))
    return pl.pallas_call(
        _node_pre_body,
        grid=grid,
        in_specs=[row, row, full, vec, vec, full, full, full],
        out_specs=[row, row, row],
        out_shape=[jax.ShapeDtypeStruct((N_AGT, D), jnp.float32)] * 3,
    )(agts, ctx, WqT, gq, bq, WBT, WCT, WaT)


_sc_mesh = plsc.VectorSubcoreMesh(core_axis_name="c", subcore_axis_name="s")


@functools.partial(
    pl.kernel,
    out_type=(
        jax.ShapeDtypeStruct((E, D), jnp.float32),
        jax.ShapeDtypeStruct((E, D), jnp.float32),
        jax.ShapeDtypeStruct((E,), jnp.float32),
        jax.ShapeDtypeStruct((E,), jnp.float32),
    ),
    mesh=_sc_mesh,
    scratch_types=[
        pltpu.VMEM((CHUNK,), jnp.int32),
        pltpu.VMEM((CHUNK,), jnp.int32),
        pltpu.VMEM((CHUNK, D), jnp.float32),
        pltpu.VMEM((CHUNK, D), jnp.float32),
        pltpu.VMEM((N_AGT,), jnp.float32),
        pltpu.VMEM((N_AGT,), jnp.float32),
        pltpu.VMEM((N_CTX,), jnp.float32),
        pltpu.VMEM((N_CTX,), jnp.float32),
        pltpu.VMEM((CHUNK,), jnp.float32),
        pltpu.VMEM((CHUNK,), jnp.float32),
        pltpu.SemaphoreType.DMA,
        pltpu.SemaphoreType.DMA,
    ],
)
def _gather_sc(hi_hbm, wi_hbm, qb_hbm, cc_hbm, xa_hbm, ya_hbm, xc_hbm, yc_hbm,
               gq_hbm, gc_hbm, dx_hbm, dy_hbm,
               hi_v, wi_v, qrows, crows, xa, ya, xc, yc, dxv, dyv, sem1, sem2):
    c = lax.axis_index("c")
    s = lax.axis_index("s")
    wid = s * NC + c
    pltpu.sync_copy(xa_hbm, xa)
    pltpu.sync_copy(ya_hbm, ya)
    pltpu.sync_copy(xc_hbm, xc)
    pltpu.sync_copy(yc_hbm, yc)
    nch = (NCHUNK - wid + NW - 1) // NW

    def body(k, carry):
        cid = wid + NW * k
        off = cid * CHUNK
        pltpu.sync_copy(hi_hbm.at[pl.ds(off, CHUNK)], hi_v)
        pltpu.sync_copy(wi_hbm.at[pl.ds(off, CHUNK)], wi_v)
        cp1 = pltpu.async_copy(qb_hbm.at[hi_v], qrows, sem1)
        cp2 = pltpu.async_copy(cc_hbm.at[wi_v], crows, sem2)
        for j in range(CHUNK // 16):
            sl = pl.ds(j * 16, 16)
            ih = hi_v[sl]
            iw = wi_v[sl]
            dxv[sl] = plsc.load_gather(xa, [ih]) - plsc.load_gather(xc, [iw])
            dyv[sl] = plsc.load_gather(ya, [ih]) - plsc.load_gather(yc, [iw])
        pltpu.sync_copy(dxv, dx_hbm.at[pl.ds(off, CHUNK)])
        pltpu.sync_copy(dyv, dy_hbm.at[pl.ds(off, CHUNK)])
        cp1.wait()
        cp2.wait()
        pltpu.sync_copy(qrows, gq_hbm.at[pl.ds(off, CHUNK)])
        pltpu.sync_copy(crows, gc_hbm.at[pl.ds(off, CHUNK)])
        return carry

    lax.fori_loop(0, nch, body, 0)


def _edge_mlp_body(dx_ref, dy_ref, gq_ref, gc_ref,
                   w1, w2, bd1, Wd2T, gd2, bd2, AT, gc1, bc1, u_ref):
    dx = dx_ref[...][:, None]
    dy = dy_ref[...][:, None]
    e1 = jnp.maximum(dx * w1[...] + dy * w2[...] + bd1[...], 0.0)
    e2 = jnp.dot(e1, Wd2T[...], preferred_element_type=jnp.float32)
    e2 = jnp.maximum(_gn(e2, gd2[...], bd2[...]), 0.0)
    y = (jnp.dot(e2, AT[...], preferred_element_type=jnp.float32)
         + gq_ref[...] + gc_ref[...])
    u_ref[...] = jnp.maximum(_gn(y, gc1[...], bc1[...]), 0.0)


def _edge_mlp(dx, dy, Gq, Gc, w1, w2, bd1, Wd2T, gd2, bd2, AT, gc1, bc1):
    grid = (E // BE,)
    vec1 = pl.BlockSpec((BE,), lambda i: (i,))
    row = pl.BlockSpec((BE, D), lambda i: (i, 0))
    full = pl.BlockSpec((D, D), lambda i: (0, 0))
    vec = pl.BlockSpec((1, D), lambda i: (0, 0))
    return pl.pallas_call(
        _edge_mlp_body,
        grid=grid,
        in_specs=[vec1, vec1, row, row, vec, vec, vec, full, vec, vec,
                  full, vec, vec],
        out_specs=row,
        out_shape=jax.ShapeDtypeStruct((E, D), jnp.float32),
    )(dx, dy, Gq, Gc, w1, w2, bd1, Wd2T, gd2, bd2, AT, gc1, bc1)


ZR = 25
ROWS_PER_SUB = N_AGT // NS
CH_PER_CORE = NCHUNK // NC


@functools.partial(
    pl.kernel,
    out_type=jax.ShapeDtypeStruct((NC, N_AGT, D), jnp.float32),
    mesh=_sc_mesh,
    scratch_types=[
        pltpu.VMEM((CHUNK,), jnp.int32),
        pltpu.VMEM((CHUNK, D), jnp.float32),
        pltpu.VMEM((ZR, D), jnp.float32),
        pltpu.VMEM_SHARED((N_AGT, D), jnp.float32),
    ],
)
def _scatter_sc(u_hbm, hi_hbm, p_hbm, hi_v, rows, zbuf, acc_sh):
    c = lax.axis_index("c")
    s = lax.axis_index("s")
    zero16 = jnp.zeros((16,), jnp.float32)
    for r in range(ZR):
        for j in range(D // 16):
            zbuf[r, pl.ds(j * 16, 16)] = zero16
    for t in range(ROWS_PER_SUB // ZR):
        pltpu.sync_copy(zbuf, acc_sh.at[pl.ds(s * ROWS_PER_SUB + t * ZR, ZR)])
    plsc.subcore_barrier()

    nch = (CH_PER_CORE - s + NS - 1) // NS

    def body(k, carry):
        cid = c * CH_PER_CORE + s + NS * k
        off = cid * CHUNK
        pltpu.sync_copy(hi_hbm.at[pl.ds(off, CHUNK)], hi_v)
        pltpu.sync_copy(u_hbm.at[pl.ds(off, CHUNK)], rows)
        pltpu.sync_copy(rows, acc_sh.at[hi_v], add=True)
        return carry

    lax.fori_loop(0, nch, body, 0)
    plsc.subcore_barrier()
    pltpu.sync_copy(acc_sh.at[pl.ds(s * ROWS_PER_SUB, ROWS_PER_SUB)],
                    p_hbm.at[c, pl.ds(s * ROWS_PER_SUB, ROWS_PER_SUB)])


def _final_body(awa_ref, p0_ref, p1_ref, agts_ref,
                Wc2T, gn_, bn_, WlT, gl_, bl_, out_ref):
    u = p0_ref[...] + p1_ref[...]
    out = awa_ref[...] + jnp.dot(u, Wc2T[...],
                                 preferred_element_type=jnp.float32)
    out = jnp.maximum(_gn(out, gn_[...], bn_[...]), 0.0)
    out = _gn(jnp.dot(out, WlT[...], preferred_element_type=jnp.float32),
              gl_[...], bl_[...])
    out_ref[...] = jnp.maximum(out + agts_ref[...], 0.0)


def _final(awa, p0, p1, agts, Wc2T, gn_, bn_, WlT, gl_, bl_):
    grid = (N_AGT // RB,)
    row = pl.BlockSpec((RB, D), lambda i: (i, 0))
    full = pl.BlockSpec((D, D), lambda i: (0, 0))
    vec = pl.BlockSpec((1, D), lambda i: (0, 0))
    return pl.pallas_call(
        _final_body,
        grid=grid,
        in_specs=[row, row, row, row, full, vec, vec, full, vec, vec],
        out_specs=row,
        out_shape=jax.ShapeDtypeStruct((N_AGT, D), jnp.float32),
    )(awa, p0, p1, agts, Wc2T, gn_, bn_, WlT, gl_, bl_)


def kernel(agts, agt_ctrs, ctx, ctx_ctrs, hi, wi,
           W_d1, b_d1, W_d2, g_d2, b_d2,
           W_q, g_q, b_q,
           W_c1, g_c1, b_c1, W_c2,
           W_a, g_n, b_n,
           W_l, g_l, b_l):
    AT = W_c1[:, :D].T
    BT = W_c1[:, D:2 * D].T
    CT = W_c1[:, 2 * D:].T
    r = lambda v: v.reshape(1, D)

    QB, CC, AWa = _node_pre(agts, ctx, W_q.T, r(g_q), r(b_q), BT, CT, W_a.T)

    Gq, Gc, dx, dy = _gather_sc(
        hi, wi, QB, CC,
        jnp.ascontiguousarray(agt_ctrs[:, 0]),
        jnp.ascontiguousarray(agt_ctrs[:, 1]),
        jnp.ascontiguousarray(ctx_ctrs[:, 0]),
        jnp.ascontiguousarray(ctx_ctrs[:, 1]))

    u = _edge_mlp(dx, dy, Gq, Gc,
                  r(W_d1[:, 0]), r(W_d1[:, 1]), r(b_d1),
                  W_d2.T, r(g_d2), r(b_d2), AT, r(g_c1), r(b_c1))

    P = _scatter_sc(u, hi)

    return _final(AWa, P[0], P[1], agts, W_c2.T,
                  r(g_n), r(b_n), W_l.T, r(g_l), r(b_l))


# speedup vs baseline: 3.7469x; 3.7469x over previous
import functools

import jax
import jax.numpy as jnp
from jax import lax
from jax.experimental import pallas as pl
from jax.experimental.pallas import tpu as pltpu
from jax.experimental.pallas import tpu_sc as plsc

N_AGT = 10000
N_CTX = 10000
E = 160000
D = 128

NC = 2
NS = 16
NW = NC * NS
CHUNK = 128
NCHUNK = E // CHUNK
RB = 400
BE = 3200
_EPS = 1e-5


def _gn(x, g, b):
    m = jnp.mean(x, axis=1, keepdims=True)
    xc = x - m
    v = jnp.mean(xc * xc, axis=1, keepdims=True)
    return xc * lax.rsqrt(v + _EPS) * g + b


def _node_pre_body(agts_ref, ctx_ref, actr_ref, cctr_ref,
                   WqT, gq, bq, WBT, WCT, WaT, w1, w2,
                   ta_ref, tc_ref, awa_ref):
    x = agts_ref[...]
    q = jnp.dot(x, WqT[...], preferred_element_type=jnp.float32)
    q = jnp.maximum(_gn(q, gq[...], bq[...]), 0.0)
    ta_ref[:, :D] = jnp.dot(q, WBT[...], preferred_element_type=jnp.float32)
    ta_ref[:, D:] = (actr_ref[:, 0:1] * w1[...] + actr_ref[:, 1:2] * w2[...])
    tc_ref[:, :D] = jnp.dot(ctx_ref[...], WCT[...],
                            preferred_element_type=jnp.float32)
    tc_ref[:, D:] = (cctr_ref[:, 0:1] * w1[...] + cctr_ref[:, 1:2] * w2[...])
    awa_ref[...] = jnp.dot(x, WaT[...], preferred_element_type=jnp.float32)


def _node_pre(agts, ctx, agt_ctrs, ctx_ctrs, WqT, gq, bq, WBT, WCT, WaT,
              w1, w2):
    grid = (N_AGT // RB,)
    row = pl.BlockSpec((RB, D), lambda i: (i, 0))
    row2 = pl.BlockSpec((RB, 2 * D), lambda i: (i, 0))
    ctr = pl.BlockSpec((RB, 2), lambda i: (i, 0))
    full = pl.BlockSpec((D, D), lambda i: (0, 0))
    vec = pl.BlockSpec((1, D), lambda i: (0, 0))
    return pl.pallas_call(
        _node_pre_body,
        grid=grid,
        in_specs=[row, row, ctr, ctr, full, vec, vec, full, full, full,
                  vec, vec],
        out_specs=[row2, row2, row],
        out_shape=[jax.ShapeDtypeStruct((N_AGT, 2 * D), jnp.float32),
                   jax.ShapeDtypeStruct((N_CTX, 2 * D), jnp.float32),
                   jax.ShapeDtypeStruct((N_AGT, D), jnp.float32)],
    )(agts, ctx, agt_ctrs, ctx_ctrs, WqT, gq, bq, WBT, WCT, WaT, w1, w2)


_sc_mesh = plsc.VectorSubcoreMesh(core_axis_name="c", subcore_axis_name="s",
                                  num_cores=NC, num_subcores=NS)


@functools.partial(
    pl.kernel,
    out_type=(
        jax.ShapeDtypeStruct((E, 2 * D), jnp.float32),
        jax.ShapeDtypeStruct((E, 2 * D), jnp.float32),
    ),
    mesh=_sc_mesh,
    scratch_types=[
        pltpu.VMEM((CHUNK,), jnp.int32),
        pltpu.VMEM((CHUNK,), jnp.int32),
        pltpu.VMEM((CHUNK, 2 * D), jnp.float32),
        pltpu.VMEM((CHUNK, 2 * D), jnp.float32),
        pltpu.SemaphoreType.DMA,
        pltpu.SemaphoreType.DMA,
    ],
)
def _gather_sc(hi_hbm, wi_hbm, ta_hbm, tb_hbm,
               ga_hbm, gx_hbm,
               hi_v, wi_v, arows, xrows, sem1, sem2):
    c = lax.axis_index("c")
    s = lax.axis_index("s")
    wid = s * NC + c
    nch = (NCHUNK - wid + NW - 1) // NW

    def body(k, carry):
        cid = wid + NW * k
        off = cid * CHUNK
        pltpu.sync_copy(hi_hbm.at[pl.ds(off, CHUNK)], hi_v)
        pltpu.sync_copy(wi_hbm.at[pl.ds(off, CHUNK)], wi_v)
        cp1 = pltpu.async_copy(ta_hbm.at[hi_v], arows, sem1)
        cp2 = pltpu.async_copy(tb_hbm.at[wi_v], xrows, sem2)
        cp1.wait()
        cp2.wait()
        pltpu.sync_copy(arows, ga_hbm.at[pl.ds(off, CHUNK)])
        pltpu.sync_copy(xrows, gx_hbm.at[pl.ds(off, CHUNK)])
        return carry

    lax.fori_loop(0, nch, body, 0)


def _edge_mlp_body(ga_ref, gx_ref,
                   bd1, Wd2T, gd2, bd2, AT, gc1, bc1, u_ref):
    e1 = jnp.maximum(ga_ref[:, D:] - gx_ref[:, D:] + bd1[...], 0.0)
    e2 = jnp.dot(e1, Wd2T[...], preferred_element_type=jnp.float32)
    e2 = jnp.maximum(_gn(e2, gd2[...], bd2[...]), 0.0)
    y = (jnp.dot(e2, AT[...], preferred_element_type=jnp.float32)
         + ga_ref[:, :D] + gx_ref[:, :D])
    u_ref[...] = jnp.maximum(_gn(y, gc1[...], bc1[...]), 0.0)


def _edge_mlp(Ga, Gx, bd1, Wd2T, gd2, bd2, AT, gc1, bc1):
    grid = (E // BE,)
    row2 = pl.BlockSpec((BE, 2 * D), lambda i: (i, 0))
    row = pl.BlockSpec((BE, D), lambda i: (i, 0))
    full = pl.BlockSpec((D, D), lambda i: (0, 0))
    vec = pl.BlockSpec((1, D), lambda i: (0, 0))
    return pl.pallas_call(
        _edge_mlp_body,
        grid=grid,
        in_specs=[row2, row2, vec, full, vec, vec, full, vec, vec],
        out_specs=row,
        out_shape=jax.ShapeDtypeStruct((E, D), jnp.float32),
    )(Ga, Gx, bd1, Wd2T, gd2, bd2, AT, gc1, bc1)


ZR = 48
RSUB = 624
TAIL = N_AGT - NS * RSUB
CH_PER_CORE = NCHUNK // NC


@functools.partial(
    pl.kernel,
    out_type=jax.ShapeDtypeStruct((NC, N_AGT, D), jnp.float32),
    mesh=_sc_mesh,
    scratch_types=[
        pltpu.VMEM((CHUNK,), jnp.int32),
        pltpu.VMEM((CHUNK, D), jnp.float32),
        pltpu.VMEM((ZR, D), jnp.float32),
        pltpu.VMEM_SHARED((N_AGT, D), jnp.float32),
    ],
)
def _scatter_sc(u_hbm, hi_hbm, p_hbm, hi_v, rows, zbuf, acc_sh):
    c = lax.axis_index("c")
    s = lax.axis_index("s")
    zero16 = jnp.zeros((16,), jnp.float32)
    for r in range(ZR):
        for j in range(D // 16):
            zbuf[r, pl.ds(j * 16, 16)] = zero16
    for t in range(RSUB // ZR):
        pltpu.sync_copy(zbuf, acc_sh.at[pl.ds(s * RSUB + t * ZR, ZR)])

    @pl.when(s == NS - 1)
    def _():
        pltpu.sync_copy(zbuf.at[pl.ds(0, TAIL)],
                        acc_sh.at[pl.ds(NS * RSUB, TAIL)])

    plsc.subcore_barrier()

    nch = (CH_PER_CORE - s + NS - 1) // NS

    def body(k, carry):
        cid = c * CH_PER_CORE + s + NS * k
        off = cid * CHUNK
        pltpu.sync_copy(hi_hbm.at[pl.ds(off, CHUNK)], hi_v)
        pltpu.sync_copy(u_hbm.at[pl.ds(off, CHUNK)], rows)
        pltpu.sync_copy(rows, acc_sh.at[hi_v], add=True)
        return carry

    lax.fori_loop(0, nch, body, 0)
    plsc.subcore_barrier()
    pltpu.sync_copy(acc_sh.at[pl.ds(s * RSUB, RSUB)],
                    p_hbm.at[c, pl.ds(s * RSUB, RSUB)])

    @pl.when(s == NS - 1)
    def _():
        pltpu.sync_copy(acc_sh.at[pl.ds(NS * RSUB, TAIL)],
                        p_hbm.at[c, pl.ds(NS * RSUB, TAIL)])


def _final_body(awa_ref, p0_ref, p1_ref, agts_ref,
                Wc2T, gn_, bn_, WlT, gl_, bl_, out_ref):
    u = p0_ref[...] + p1_ref[...]
    out = awa_ref[...] + jnp.dot(u, Wc2T[...],
                                 preferred_element_type=jnp.float32)
    out = jnp.maximum(_gn(out, gn_[...], bn_[...]), 0.0)
    out = _gn(jnp.dot(out, WlT[...], preferred_element_type=jnp.float32),
              gl_[...], bl_[...])
    out_ref[...] = jnp.maximum(out + agts_ref[...], 0.0)


def _final(awa, p0, p1, agts, Wc2T, gn_, bn_, WlT, gl_, bl_):
    grid = (N_AGT // RB,)
    row = pl.BlockSpec((RB, D), lambda i: (i, 0))
    full = pl.BlockSpec((D, D), lambda i: (0, 0))
    vec = pl.BlockSpec((1, D), lambda i: (0, 0))
    return pl.pallas_call(
        _final_body,
        grid=grid,
        in_specs=[row, row, row, row, full, vec, vec, full, vec, vec],
        out_specs=row,
        out_shape=jax.ShapeDtypeStruct((N_AGT, D), jnp.float32),
    )(awa, p0, p1, agts, Wc2T, gn_, bn_, WlT, gl_, bl_)


def kernel(agts, agt_ctrs, ctx, ctx_ctrs, hi, wi,
           W_d1, b_d1, W_d2, g_d2, b_d2,
           W_q, g_q, b_q,
           W_c1, g_c1, b_c1, W_c2,
           W_a, g_n, b_n,
           W_l, g_l, b_l):
    AT = W_c1[:, :D].T
    BT = W_c1[:, D:2 * D].T
    CT = W_c1[:, 2 * D:].T
    r = lambda v: v.reshape(1, D)

    TA, TB, AWa = _node_pre(agts, ctx, agt_ctrs, ctx_ctrs,
                            W_q.T, r(g_q), r(b_q), BT, CT, W_a.T,
                            r(W_d1[:, 0]), r(W_d1[:, 1]))

    Ga, Gx = _gather_sc(hi, wi, TA, TB)

    u = _edge_mlp(Ga, Gx, r(b_d1),
                  W_d2.T, r(g_d2), r(b_d2), AT, r(g_c1), r(b_c1))

    P = _scatter_sc(u, hi)

    return _final(AWa, P[0], P[1], agts, W_c2.T,
                  r(g_n), r(b_n), W_l.T, r(g_l), r(b_l))
